# baseline (device time: 56324 ns/iter reference)
import jax
import jax.numpy as jnp
from jax import lax
from jax.experimental import pallas as pl
from jax.experimental.pallas import tpu as pltpu

B = 4
S = 512
S_HALF = S // 2
K = 512
N = 1024


def kernel(O, Wo):
    O3 = O.reshape(B, S, K)

    def body(o_ref, w_ref, out_ref, send_buf, recv_buf, send_sem, recv_sem):
        my_x = lax.axis_index("x")
        my_y = lax.axis_index("y")
        my_z = lax.axis_index("z")
        nbr = (1 - my_x, my_y, my_z)

        barrier_sem = pltpu.get_barrier_semaphore()
        pl.semaphore_signal(
            barrier_sem, inc=1, device_id=nbr,
            device_id_type=pl.DeviceIdType.MESH,
        )
        pl.semaphore_wait(barrier_sem, 1)

        w = w_ref[:, :]

        nbr_s0 = (1 - my_x) * S_HALF
        for b in range(B):
            send_buf[b, :, :] = jnp.dot(
                o_ref[b, pl.ds(nbr_s0, S_HALF), :], w,
                preferred_element_type=jnp.float32,
            )
        rdma = pltpu.make_async_remote_copy(
            src_ref=send_buf,
            dst_ref=recv_buf,
            send_sem=send_sem,
            recv_sem=recv_sem,
            device_id=nbr,
            device_id_type=pl.DeviceIdType.MESH,
        )
        rdma.start()

        my_s0 = my_x * S_HALF
        for b in range(B):
            out_ref[b, :, :] = jnp.dot(
                o_ref[b, pl.ds(my_s0, S_HALF), :], w,
                preferred_element_type=jnp.float32,
            )

        rdma.wait()
        for b in range(B):
            out_ref[b, :, :] += recv_buf[b, :, :]

    return pl.pallas_call(
        body,
        out_shape=jax.ShapeDtypeStruct((B, S_HALF, N), jnp.float32),
        in_specs=[
            pl.BlockSpec(memory_space=pltpu.VMEM),
            pl.BlockSpec(memory_space=pltpu.VMEM),
        ],
        out_specs=pl.BlockSpec(memory_space=pltpu.VMEM),
        scratch_shapes=[
            pltpu.VMEM((B, S_HALF, N), jnp.float32),
            pltpu.VMEM((B, S_HALF, N), jnp.float32),
            pltpu.SemaphoreType.DMA,
            pltpu.SemaphoreType.DMA,
        ],
        compiler_params=pltpu.CompilerParams(collective_id=0),
    )(O3, Wo)


# device time: 55036 ns/iter; 1.0234x vs baseline; 1.0234x over previous
import jax
import jax.numpy as jnp
from jax import lax
from jax.experimental import pallas as pl
from jax.experimental.pallas import tpu as pltpu

B = 4
S = 512
S_HALF = S // 2
K = 512
N = 1024
CHUNKS_PER_B = 2
S_CHUNK = S_HALF // CHUNKS_PER_B
N_CHUNKS = B * CHUNKS_PER_B


def kernel(O, Wo):
    O3 = O.reshape(B, S, K)

    def body(o_ref, w_ref, out_ref, send_buf, recv_buf, send_sems, recv_sems):
        my_x = lax.axis_index("x")
        my_y = lax.axis_index("y")
        my_z = lax.axis_index("z")
        nbr = (1 - my_x, my_y, my_z)

        barrier_sem = pltpu.get_barrier_semaphore()
        pl.semaphore_signal(
            barrier_sem, inc=1, device_id=nbr,
            device_id_type=pl.DeviceIdType.MESH,
        )
        pl.semaphore_wait(barrier_sem, 1)

        w = w_ref[:, :]
        nbr_s0 = (1 - my_x) * S_HALF
        my_s0 = my_x * S_HALF

        def chunk_rdma(b, j):
            c = b * CHUNKS_PER_B + j
            return pltpu.make_async_remote_copy(
                src_ref=send_buf.at[b, pl.ds(j * S_CHUNK, S_CHUNK)],
                dst_ref=recv_buf.at[b, pl.ds(j * S_CHUNK, S_CHUNK)],
                send_sem=send_sems.at[c],
                recv_sem=recv_sems.at[c],
                device_id=nbr,
                device_id_type=pl.DeviceIdType.MESH,
            )

        for b in range(B):
            for j in range(CHUNKS_PER_B):
                send_buf[b, pl.ds(j * S_CHUNK, S_CHUNK), :] = jnp.dot(
                    o_ref[b, pl.ds(nbr_s0 + j * S_CHUNK, S_CHUNK), :], w,
                    preferred_element_type=jnp.float32,
                )
                chunk_rdma(b, j).start()

        for b in range(B):
            out_ref[b, :, :] = jnp.dot(
                o_ref[b, pl.ds(my_s0, S_HALF), :], w,
                preferred_element_type=jnp.float32,
            )

        for b in range(B):
            for j in range(CHUNKS_PER_B):
                chunk_rdma(b, j).wait_recv()
                sl = pl.ds(j * S_CHUNK, S_CHUNK)
                out_ref[b, sl, :] += recv_buf[b, sl, :]

        for b in range(B):
            for j in range(CHUNKS_PER_B):
                chunk_rdma(b, j).wait_send()

    return pl.pallas_call(
        body,
        out_shape=jax.ShapeDtypeStruct((B, S_HALF, N), jnp.float32),
        in_specs=[
            pl.BlockSpec(memory_space=pltpu.VMEM),
            pl.BlockSpec(memory_space=pltpu.VMEM),
        ],
        out_specs=pl.BlockSpec(memory_space=pltpu.VMEM),
        scratch_shapes=[
            pltpu.VMEM((B, S_HALF, N), jnp.float32),
            pltpu.VMEM((B, S_HALF, N), jnp.float32),
            pltpu.SemaphoreType.DMA((N_CHUNKS,)),
            pltpu.SemaphoreType.DMA((N_CHUNKS,)),
        ],
        compiler_params=pltpu.CompilerParams(collective_id=0),
    )(O3, Wo)
